# Initial kernel scaffold; baseline (speedup 1.0000x reference)
import jax
import jax.numpy as jnp
from jax.experimental import pallas as pl


def kernel(hidden_states, gate_w, w_gate, w_up, w_down, mlp_buffer, gathered_experts_out_buf):
    raise NotImplementedError("write your pallas kernel here")


# speedup vs baseline: 2.5761x; 2.5761x over previous
import jax
import jax.numpy as jnp
from jax.experimental import pallas as pl
from jax.experimental.pallas import tpu as pltpu

K_TOP = 2


def _moe_kernel(x_ref, xb_ref, gw_ref, wg_ref, wu_ref, wd_ref, o_ref, w_ref):
    e = pl.program_id(0)

    @pl.when(e == 0)
    def _router():
        x = x_ref[...]
        logits = jnp.dot(x, gw_ref[...].T,
                         preferred_element_type=jnp.float32)
        m = jnp.max(logits, axis=-1, keepdims=True)
        ex = jnp.exp(logits - m)
        p = ex / jnp.sum(ex, axis=-1, keepdims=True)
        num_experts = p.shape[-1]
        idx = jax.lax.broadcasted_iota(jnp.int32, p.shape, 1)
        rank = jnp.zeros(p.shape, dtype=jnp.int32)
        for j in range(num_experts):
            pj = p[:, j:j + 1]
            beats = (pj > p) | ((pj == p) & (j < idx))
            rank = rank + beats.astype(jnp.int32)
        sel = rank < K_TOP
        w = jnp.where(sel, p, 0.0)
        w = w / jnp.sum(w, axis=-1, keepdims=True)
        w_ref[...] = w

    xb = xb_ref[...]
    w_all = w_ref[...]
    lane = jax.lax.broadcasted_iota(jnp.int32, w_all.shape, 1)
    we = jnp.sum(jnp.where(lane == e, w_all, 0.0),
                 axis=1, keepdims=True)
    g = jnp.dot(xb, wg_ref[0], preferred_element_type=jnp.float32)
    u = jnp.dot(xb, wu_ref[0], preferred_element_type=jnp.float32)
    h = (g * jax.lax.logistic(g)) * u
    hw = (h * we).astype(jnp.bfloat16)
    y = jnp.dot(hw, wd_ref[0], preferred_element_type=jnp.float32)

    @pl.when(e == 0)
    def _init():
        o_ref[...] = y

    @pl.when(e != 0)
    def _acc():
        o_ref[...] += y


def kernel(hidden_states, gate_w, w_gate, w_up, w_down,
           mlp_buffer=None, gathered_experts_out_buf=None):
    T, D = hidden_states.shape[0], hidden_states.shape[-1]
    E = gate_w.shape[0]
    F = w_gate.shape[-1]
    x = hidden_states.reshape(T, D)
    xb = x.astype(jnp.bfloat16)
    wgb = w_gate.astype(jnp.bfloat16)
    wub = w_up.astype(jnp.bfloat16)
    wdb = w_down.astype(jnp.bfloat16)

    out = pl.pallas_call(
        _moe_kernel,
        grid=(E,),
        in_specs=[
            pl.BlockSpec((T, D), lambda e: (0, 0)),
            pl.BlockSpec((T, D), lambda e: (0, 0)),
            pl.BlockSpec((E, D), lambda e: (0, 0)),
            pl.BlockSpec((1, D, F), lambda e: (e, 0, 0)),
            pl.BlockSpec((1, D, F), lambda e: (e, 0, 0)),
            pl.BlockSpec((1, F, D), lambda e: (e, 0, 0)),
        ],
        out_specs=pl.BlockSpec((T, D), lambda e: (0, 0)),
        out_shape=jax.ShapeDtypeStruct((T, D), jnp.float32),
        scratch_shapes=[pltpu.VMEM((T, E), jnp.float32)],
    )(x, xb, gate_w, wgb, wub, wdb)
    return out.reshape(hidden_states.shape)


# speedup vs baseline: 3.2537x; 1.2630x over previous
import jax
import jax.numpy as jnp
from jax.experimental import pallas as pl
from jax.experimental.pallas import tpu as pltpu

K_TOP = 2


def _moe_kernel(x_ref, xb_ref, gw_ref, wg_ref, wu_ref, wd_ref, o_ref, w_ref):
    e = pl.program_id(0)

    @pl.when(e == 0)
    def _router():
        x = x_ref[...]
        logits = jnp.dot(x, gw_ref[...].T,
                         preferred_element_type=jnp.float32)
        m = jnp.max(logits, axis=-1, keepdims=True)
        ex = jnp.exp(logits - m)
        p = ex / jnp.sum(ex, axis=-1, keepdims=True)
        num_experts = p.shape[-1]
        idx = jax.lax.broadcasted_iota(jnp.int32, p.shape, 1)
        rank = jnp.zeros(p.shape, dtype=jnp.int32)
        for j in range(num_experts):
            pj = p[:, j:j + 1]
            beats = (pj > p) | ((pj == p) & (j < idx))
            rank = rank + beats.astype(jnp.int32)
        sel = rank < K_TOP
        w = jnp.where(sel, p, 0.0)
        w = w / jnp.sum(w, axis=-1, keepdims=True)
        w_ref[...] = w

    xb = xb_ref[...]
    w_all = w_ref[...]
    lane = jax.lax.broadcasted_iota(jnp.int32, w_all.shape, 1)
    we = jnp.sum(jnp.where(lane == e, w_all, 0.0),
                 axis=1, keepdims=True)
    wg = wg_ref[0].astype(jnp.bfloat16)
    wu = wu_ref[0].astype(jnp.bfloat16)
    wd = wd_ref[0].astype(jnp.bfloat16)
    g = jnp.dot(xb, wg, preferred_element_type=jnp.float32)
    u = jnp.dot(xb, wu, preferred_element_type=jnp.float32)
    h = (g * jax.lax.logistic(g)) * u
    hw = (h * we).astype(jnp.bfloat16)
    y = jnp.dot(hw, wd, preferred_element_type=jnp.float32)

    @pl.when(e == 0)
    def _init():
        o_ref[...] = y

    @pl.when(e != 0)
    def _acc():
        o_ref[...] += y


def kernel(hidden_states, gate_w, w_gate, w_up, w_down,
           mlp_buffer=None, gathered_experts_out_buf=None):
    T, D = hidden_states.shape[0], hidden_states.shape[-1]
    E = gate_w.shape[0]
    F = w_gate.shape[-1]
    x = hidden_states.reshape(T, D)
    xb = x.astype(jnp.bfloat16)

    out = pl.pallas_call(
        _moe_kernel,
        grid=(E,),
        in_specs=[
            pl.BlockSpec((T, D), lambda e: (0, 0)),
            pl.BlockSpec((T, D), lambda e: (0, 0)),
            pl.BlockSpec((E, D), lambda e: (0, 0)),
            pl.BlockSpec((1, D, F), lambda e: (e, 0, 0)),
            pl.BlockSpec((1, D, F), lambda e: (e, 0, 0)),
            pl.BlockSpec((1, F, D), lambda e: (e, 0, 0)),
        ],
        out_specs=pl.BlockSpec((T, D), lambda e: (0, 0)),
        out_shape=jax.ShapeDtypeStruct((T, D), jnp.float32),
        scratch_shapes=[pltpu.VMEM((T, E), jnp.float32)],
    )(x, xb, gate_w, w_gate, w_up, w_down)
    return out.reshape(hidden_states.shape)
